# Initial kernel scaffold; baseline (speedup 1.0000x reference)
import jax
import jax.numpy as jnp
from jax.experimental import pallas as pl


def kernel(z_e, embed_weight):
    raise NotImplementedError("write your pallas kernel here")


# speedup vs baseline: 1.1961x; 1.1961x over previous
import functools

import jax
import jax.numpy as jnp
from jax import lax
from jax.experimental import pallas as pl
from jax.experimental.pallas import tpu as pltpu
from jax.experimental.pallas import tpu_sc as plsc

_K = 8192
_D = 256
_N = 16384
_TR = 256
_NT = _N // _TR


_WLO = (0, 2736, 5472)
_WHI = (2736, 5472, _K)


def _argmin_body(flat_ref, emb_ref, fn_ref, en_ref, idx_ref, loss_ref):
    i = pl.program_id(0)

    @pl.when(i == 0)
    def _():
        loss_ref[0, 0] = 0.0

    flat = flat_ref[...]
    fn = fn_ref[...][:, None]
    mm = lax.dot_general(
        flat.astype(jnp.bfloat16), emb_ref[...].astype(jnp.bfloat16),
        (((1,), (1,)), ((), ())),
        preferred_element_type=jnp.float32)
    dist = (fn + en_ref[...][None, :]) - 2.0 * mm

    iota_c = lax.broadcasted_iota(jnp.int32, (1, _K), 1)
    inf = jnp.float32(jnp.inf)
    acc_v = None
    for w in range(3):
        in_w = (iota_c >= _WLO[w]) & (iota_c < _WHI[w])
        m = jnp.min(jnp.where(in_w, dist, inf), axis=1)
        if acc_v is None:
            sel_v, sel_w = m, jnp.zeros_like(m, jnp.int32)
            acc_v = m.astype(jnp.bfloat16).astype(jnp.float32)
        else:
            pred = m < acc_v
            sel_v = jnp.where(pred, m, sel_v)
            sel_w = jnp.where(pred, w, sel_w)
            acc_v = jnp.where(pred, m, acc_v).astype(jnp.bfloat16).astype(jnp.float32)

    wcol = (iota_c >= _WLO[1]).astype(jnp.int32) + (iota_c >= _WLO[2]).astype(jnp.int32)
    hit = (dist == sel_v[:, None]) & (wcol == sel_w[:, None])
    idx = jnp.min(jnp.where(hit, jnp.broadcast_to(iota_c, (_TR, _K)), _K), axis=1)
    idx_ref[...] = idx
    loss_ref[0, 0] += jnp.sum(sel_v)


_argmin_call = pl.pallas_call(
    _argmin_body,
    grid=(_NT,),
    in_specs=[
        pl.BlockSpec((_TR, _D), lambda i: (i, 0)),
        pl.BlockSpec((_K, _D), lambda i: (0, 0)),
        pl.BlockSpec((_TR,), lambda i: (i,)),
        pl.BlockSpec((_K,), lambda i: (0,)),
    ],
    out_specs=[
        pl.BlockSpec((_TR,), lambda i: (i,)),
        pl.BlockSpec(memory_space=pltpu.SMEM),
    ],
    out_shape=[
        jax.ShapeDtypeStruct((_N,), jnp.int32),
        jax.ShapeDtypeStruct((1, 1), jnp.float32),
    ],
)


_NC, _NS = 2, 16
_NW = _NC * _NS
_BPW = _N // _NW
_CH = 128
_NCH = _BPW // _CH


@functools.cache
def _sc_gather_call():
    @functools.partial(
        pl.kernel,
        mesh=plsc.VectorSubcoreMesh(core_axis_name="c", subcore_axis_name="s"),
        out_type=jax.ShapeDtypeStruct((_N, _D), jnp.float32),
        scratch_types=[
            pltpu.VMEM((_NCH, _CH), jnp.int32),
            pltpu.VMEM((_CH, _D), jnp.float32),
            pltpu.SemaphoreType.DMA,
        ],
    )
    def _sc_gather(table_hbm, idx_hbm, out_hbm, idx_v, rows_v, sem):
        wid = lax.axis_index("s") * _NC + lax.axis_index("c")
        base = wid * _BPW
        for c in range(_NCH):
            pltpu.sync_copy(idx_hbm.at[pl.ds(base + c * _CH, _CH)], idx_v.at[c])
            pltpu.async_copy(table_hbm.at[idx_v.at[c]], rows_v, sem).wait()
            pltpu.sync_copy(rows_v, out_hbm.at[pl.ds(base + c * _CH, _CH)])

    return _sc_gather


def kernel(z_e, embed_weight):
    flat = z_e.reshape(_N, _D)
    fn = jnp.sum(flat ** 2, axis=1)
    en = jnp.sum(embed_weight ** 2, axis=1)
    idx, loss_sum = _argmin_call(flat, embed_weight, fn, en)
    quantized = _sc_gather_call()(embed_weight, idx)
    m = loss_sum[0, 0] / (_N * _D)
    loss = m + 0.25 * m
    return quantized.reshape(z_e.shape), loss, idx
